# baseline (speedup 1.0000x reference)
import functools

import jax
import jax.numpy as jnp
from jax.experimental import pallas as pl
from jax.experimental.pallas import tpu as pltpu

_ROWS = 400


def _stats_body(h_ref, g_ref, colsum_ref, min_ref):
    step = pl.program_id(0)

    @pl.when(step == 0)
    def _init():
        colsum_ref[...] = jnp.zeros_like(colsum_ref)
        min_ref[...] = jnp.full_like(min_ref, jnp.inf)

    h = h_ref[...]
    g = g_ref[...]
    g = jnp.where(jnp.isnan(g), 0.0, g)
    colsum_ref[...] += jnp.sum(h.reshape(_ROWS // 8, 8, h.shape[1]), axis=0)
    min_ref[...] = jnp.minimum(min_ref[...], jnp.min(g))


def _valid_body(h_ref, g_ref, colsum_ref, min_ref, out_ref):
    h = h_ref[...]
    g = g_ref[...]
    g = jnp.where(jnp.isnan(g), 0.0, g)
    gmin = min_ref[0, 0]
    edeg_le2 = colsum_ref[0, :] <= 2.0
    vdeg_le1 = jnp.sum(h, axis=1, keepdims=True) <= 1.0
    l_and = jnp.where(vdeg_le1 | edeg_le2[None, :], h, 0.0)
    out_ref[...] = (g - gmin) * (1.0 - l_and)


def _valid_gradients(H, gradients):
    n, e = H.shape
    grid = (n // _ROWS,)
    colsum8, min8 = pl.pallas_call(
        _stats_body,
        grid=grid,
        in_specs=[
            pl.BlockSpec((_ROWS, e), lambda i: (i, 0)),
            pl.BlockSpec((_ROWS, e), lambda i: (i, 0)),
        ],
        out_specs=[
            pl.BlockSpec((8, e), lambda i: (0, 0)),
            pl.BlockSpec((8, 128), lambda i: (0, 0)),
        ],
        out_shape=[
            jax.ShapeDtypeStruct((8, e), jnp.float32),
            jax.ShapeDtypeStruct((8, 128), jnp.float32),
        ],
    )(H, gradients)
    colsum = jnp.sum(colsum8, axis=0, keepdims=True)
    gmin = jnp.min(min8, keepdims=True)

    valid = pl.pallas_call(
        _valid_body,
        grid=grid,
        in_specs=[
            pl.BlockSpec((_ROWS, e), lambda i: (i, 0)),
            pl.BlockSpec((_ROWS, e), lambda i: (i, 0)),
            pl.BlockSpec((1, e), lambda i: (0, 0)),
            pl.BlockSpec((1, 128), lambda i: (0, 0)),
        ],
        out_specs=pl.BlockSpec((_ROWS, e), lambda i: (i, 0)),
        out_shape=jax.ShapeDtypeStruct((n, e), jnp.float32),
    )(H, gradients, colsum, gmin.reshape(1, 1) * jnp.ones((1, 128), jnp.float32))
    return valid


def kernel(H, gradients):
    valid = _valid_gradients(H, gradients)
    sorted_idx = jnp.argsort(-valid.reshape(-1))
    return valid, sorted_idx
